# baseline (speedup 1.0000x reference)
import functools

import jax
import jax.numpy as jnp
from jax import lax
from jax.experimental import pallas as pl
from jax.experimental.pallas import tpu as pltpu
from jax.experimental.pallas import tpu_sc as plsc

_N, _D, _H, _O, _E, _K = 2048, 1024, 1024, 1024, 8, 2
_S = _N * _K
_T = 256
_NT = (_S + _E * _T) // _T
_PAD = _NT * _T
_EP = 128

_NC, _NS = 2, 16
_NW = _NC * _NS
_DROWS = _PAD // _NW
_DCH = 48
_TOKW = _N // _NW
_CCH = 16


def _gating_body(x_ref, wg_ref, idx_ref, g_ref, loss_ref):
    x = x_ref[...]
    wg = wg_ref[...]
    logits = jnp.dot(x, wg, preferred_element_type=jnp.float32)
    lane = lax.broadcasted_iota(jnp.int32, (_N, _EP), 1)
    neg = jnp.float32(-jnp.inf)
    logits = jnp.where(lane < _E, logits, neg)
    m1 = jnp.max(logits, axis=1, keepdims=True)
    i1 = jnp.min(jnp.where(logits == m1, lane, _EP), axis=1, keepdims=True)
    masked = jnp.where(lane == i1, neg, logits)
    m2 = jnp.max(masked, axis=1, keepdims=True)
    i2 = jnp.min(jnp.where(masked == m2, lane, _EP), axis=1, keepdims=True)
    t = jnp.exp(m2 - m1)
    g1 = 1.0 / (1.0 + t)
    g2 = t / (1.0 + t)
    gates = jnp.where(lane == i1, g1, 0.0) + jnp.where(lane == i2, g2, 0.0)
    gates = jnp.where(lane < _E, gates, 0.0)
    idx_ref[...] = jnp.where(lane == 0, i1, jnp.where(lane == 1, i2, 0))
    g_ref[...] = jnp.where(lane == 0, g1, jnp.where(lane == 1, g2, 0.0))
    emask = (lane[:1] < _E).astype(jnp.float32)
    imp = jnp.sum(gates, axis=0, keepdims=True)
    load = jnp.sum((gates > 0).astype(jnp.float32), axis=0, keepdims=True)

    def cv2(v):
        m = jnp.sum(v * emask) / _E
        var = jnp.sum((v - m) ** 2 * emask) / (_E - 1)
        return var / (m * m + 1e-10)

    loss = (cv2(imp) + cv2(load)) * 0.01
    loss_ref[...] = jnp.full((8, 128), loss, dtype=jnp.float32)


def _gating_call(x, wg_pad, interpret=False):
    return pl.pallas_call(
        _gating_body,
        out_shape=(
            jax.ShapeDtypeStruct((_N, _EP), jnp.int32),
            jax.ShapeDtypeStruct((_N, _EP), jnp.float32),
            jax.ShapeDtypeStruct((8, 128), jnp.float32),
        ),
        interpret=interpret,
    )(x, wg_pad)


def _gmm_body(meta_ref, xs_ref, w1_ref, b1_ref, w2_ref, b2_ref, gp_ref, out_ref):
    t = pl.program_id(0)

    @pl.when(meta_ref[2, t] == 1)
    def _():
        xk = xs_ref[...]
        h = jnp.dot(xk, w1_ref[0], preferred_element_type=jnp.float32)
        h = jnp.maximum(h + b1_ref[0], 0.0).astype(jnp.bfloat16)
        o = jnp.dot(h, w2_ref[0], preferred_element_type=jnp.float32)
        o = o + b2_ref[0]
        out_ref[...] = o * gp_ref[0, 0][:, None]


def _gmm_call(meta, xs, w1, b1, w2, b2, gp3, interpret=False):
    grid_spec = pltpu.PrefetchScalarGridSpec(
        num_scalar_prefetch=1,
        grid=(_NT,),
        in_specs=[
            pl.BlockSpec((_T, _D), lambda t, m: (m[1, t], 0)),
            pl.BlockSpec((1, _D, _H), lambda t, m: (m[0, t], 0, 0)),
            pl.BlockSpec((1, 1, _H), lambda t, m: (m[0, t], 0, 0)),
            pl.BlockSpec((1, _H, _O), lambda t, m: (m[0, t], 0, 0)),
            pl.BlockSpec((1, 1, _O), lambda t, m: (m[0, t], 0, 0)),
            pl.BlockSpec((1, 1, _T), lambda t, m: (m[1, t], 0, 0)),
        ],
        out_specs=pl.BlockSpec((_T, _O), lambda t, m: (m[1, t], 0)),
    )
    return pl.pallas_call(
        _gmm_body,
        grid_spec=grid_spec,
        out_shape=jax.ShapeDtypeStruct((_PAD, _O), jnp.float32),
        interpret=interpret,
    )(meta, xs, w1, b1, w2, b2, gp3)


@functools.lru_cache(maxsize=None)
def _sc_kernels():
    mesh = plsc.VectorSubcoreMesh(core_axis_name="c", subcore_axis_name="s")

    nch = _DROWS // _DCH

    @functools.partial(
        pl.kernel,
        out_type=jax.ShapeDtypeStruct((_PAD, _D // 2), jnp.int32),
        mesh=mesh,
        scratch_types=(
            [pltpu.VMEM((_DROWS,), jnp.int32)]
            + [pltpu.VMEM((_DCH, _D // 2), jnp.int32) for _ in range(nch)]
            + [pltpu.SemaphoreType.DMA for _ in range(2 * nch)]
        ),
    )
    def _sc_dispatch(x_hbm, tok_hbm, out_hbm, idx_v, *rest):
        bufs = rest[:nch]
        gsem = rest[nch:2 * nch]
        wsem = rest[2 * nch:]
        wid = lax.axis_index("s") * _NC + lax.axis_index("c")
        base = wid * _DROWS
        pltpu.sync_copy(tok_hbm.at[pl.ds(base, _DROWS)], idx_v)
        gathers = [
            pltpu.async_copy(
                x_hbm.at[idx_v.at[pl.ds(c * _DCH, _DCH)]], bufs[c], gsem[c])
            for c in range(nch)
        ]
        writes = [None] * nch
        for c in range(nch):
            gathers[c].wait()
            writes[c] = pltpu.async_copy(
                bufs[c], out_hbm.at[pl.ds(base + c * _DCH, _DCH)], wsem[c])
        for c in range(nch):
            writes[c].wait()

    @functools.partial(
        pl.kernel,
        out_type=jax.ShapeDtypeStruct((_N, _O), jnp.float32),
        mesh=mesh,
        scratch_types=[
            pltpu.VMEM((_TOKW,), jnp.int32),
            pltpu.VMEM((_TOKW,), jnp.int32),
            pltpu.VMEM((_CCH, _O), jnp.float32),
            pltpu.VMEM((_CCH, _O), jnp.float32),
            pltpu.VMEM((_CCH, _O), jnp.float32),
            pltpu.VMEM((_CCH, _O), jnp.float32),
            pltpu.SemaphoreType.DMA,
            pltpu.SemaphoreType.DMA,
            pltpu.SemaphoreType.DMA,
            pltpu.SemaphoreType.DMA,
            pltpu.SemaphoreType.DMA,
            pltpu.SemaphoreType.DMA,
        ],
    )
    def _sc_combine(rows_hbm, pos0_hbm, pos1_hbm, y_hbm,
                    i0_v, i1_v, a0, a1, c0, c1,
                    ga0, ga1, gb0, gb1, wa, wb):
        wid = lax.axis_index("s") * _NC + lax.axis_index("c")
        base = wid * _TOKW
        pltpu.sync_copy(pos0_hbm.at[pl.ds(base, _TOKW)], i0_v)
        pltpu.sync_copy(pos1_hbm.at[pl.ds(base, _TOKW)], i1_v)
        slots = ((a0, a1, ga0, ga1, wa), (c0, c1, gb0, gb1, wb))
        nch = _TOKW // _CCH
        gathers = [None] * nch
        writes = [None] * nch

        def start(c):
            r0, r1, gs0, gs1, _ = slots[c % 2]
            sl = pl.ds(c * _CCH, _CCH)
            cp0 = pltpu.async_copy(rows_hbm.at[i0_v.at[sl]], r0, gs0)
            cp1 = pltpu.async_copy(rows_hbm.at[i1_v.at[sl]], r1, gs1)
            return (cp0, cp1)

        gathers[0] = start(0)
        for c in range(nch):
            r0, r1, _, _, ws = slots[c % 2]
            gathers[c][0].wait()
            gathers[c][1].wait()
            if c + 1 < nch:
                if c >= 1:
                    writes[c - 1].wait()
                gathers[c + 1] = start(c + 1)

            def row_body(r, _):
                def vec_body(v, _):
                    sl = pl.ds(v * 16, 16)
                    r0[r, sl] = r0[r, sl] + r1[r, sl]
                    return 0

                return lax.fori_loop(0, _O // 16, vec_body, 0)

            lax.fori_loop(0, _CCH, row_body, 0)
            writes[c] = pltpu.async_copy(
                r0, y_hbm.at[pl.ds(base + c * _CCH, _CCH)], ws)
        if nch >= 2:
            writes[nch - 2].wait()
        writes[nch - 1].wait()

    return _sc_dispatch, _sc_combine


def _routing(i12, g12):
    e_flat = i12.reshape(-1)
    g_flat = g12.reshape(-1)
    order = jnp.argsort(e_flat, stable=True)
    e_sorted = e_flat[order]
    counts = jnp.bincount(e_flat, length=_E).astype(jnp.int32)
    tiles = (counts + _T - 1) // _T
    zero1 = jnp.zeros((1,), jnp.int32)
    pad_start = jnp.concatenate([zero1, jnp.cumsum(tiles * _T)])
    start = jnp.concatenate([zero1, jnp.cumsum(counts)])
    j = jnp.arange(_S, dtype=jnp.int32)
    dest = pad_start[e_sorted] + (j - start[e_sorted])
    tok_flat = jnp.arange(_S, dtype=jnp.int32) // _K
    tok_pad = jnp.zeros((_PAD,), jnp.int32).at[dest].set(tok_flat[order])
    g_pad = jnp.zeros((_PAD,), jnp.float32).at[dest].set(g_flat[order])
    pos_flat = jnp.zeros((_S,), jnp.int32).at[order].set(dest)
    pos = pos_flat.reshape(_N, _K)
    nv = pad_start[_E] // _T
    tvec = jnp.arange(_NT, dtype=jnp.int32)
    eid = jnp.clip(jnp.searchsorted(pad_start[1:], tvec * _T, side="right"),
                   0, _E - 1).astype(jnp.int32)
    eid = jnp.where(tvec < nv, eid, eid[nv - 1])
    blk = jnp.where(tvec < nv, tvec, nv - 1)
    valid = (tvec < nv).astype(jnp.int32)
    meta = jnp.stack([eid, blk, valid])
    return tok_pad, g_pad, pos, meta


def kernel(x, w_gate, W1, b1, W2, b2):
    wg_pad = jnp.zeros((_D, _EP), jnp.float32).at[:, :_E].set(w_gate)
    idx8, g8, loss8 = _gating_call(x, wg_pad)
    i12 = idx8[:, :_K]
    g12 = g8[:, :_K]
    tok_pad, g_pad, pos, meta = _routing(i12, g12)
    sc_dispatch, sc_combine = _sc_kernels()
    xb = x.astype(jnp.bfloat16)
    x_u32 = lax.bitcast_convert_type(xb.reshape(_N, _D // 2, 2), jnp.int32)
    xs_u32 = sc_dispatch(x_u32, tok_pad)
    xs = lax.bitcast_convert_type(xs_u32, jnp.bfloat16).reshape(_PAD, _D)
    gp3 = g_pad.reshape(_NT, 1, _T)
    rows = _gmm_call(meta, xs, W1.astype(jnp.bfloat16), b1.reshape(_E, 1, _H),
                     W2.astype(jnp.bfloat16), b2.reshape(_E, 1, _O), gp3)
    y = sc_combine(rows, pos[:, 0], pos[:, 1])
    return y, loss8[0, 0]


# speedup vs baseline: 1.4817x; 1.4817x over previous
import functools

import jax
import jax.numpy as jnp
from jax import lax
from jax.experimental import pallas as pl
from jax.experimental.pallas import tpu as pltpu
from jax.experimental.pallas import tpu_sc as plsc

_N, _D, _H, _O, _E, _K = 2048, 1024, 1024, 1024, 8, 2
_S = _N * _K
_T = 256
_NT = (_S + _E * _T) // _T
_PAD = _NT * _T
_EP = 128

_NC, _NS = 2, 16
_NW = _NC * _NS
_DROWS = _PAD // _NW
_DCH = 48
_DSR = 16
_TOKW = _N // _NW
_CCH = 16


def _gating_body(x_ref, wg_ref, idx_ref, g_ref, loss_ref):
    x = x_ref[...]
    wg = wg_ref[...]
    logits = jnp.dot(x, wg, preferred_element_type=jnp.float32)
    lane = lax.broadcasted_iota(jnp.int32, (_N, _EP), 1)
    neg = jnp.float32(-jnp.inf)
    logits = jnp.where(lane < _E, logits, neg)
    m1 = jnp.max(logits, axis=1, keepdims=True)
    i1 = jnp.min(jnp.where(logits == m1, lane, _EP), axis=1, keepdims=True)
    masked = jnp.where(lane == i1, neg, logits)
    m2 = jnp.max(masked, axis=1, keepdims=True)
    i2 = jnp.min(jnp.where(masked == m2, lane, _EP), axis=1, keepdims=True)
    t = jnp.exp(m2 - m1)
    g1 = 1.0 / (1.0 + t)
    g2 = t / (1.0 + t)
    gates = jnp.where(lane == i1, g1, 0.0) + jnp.where(lane == i2, g2, 0.0)
    gates = jnp.where(lane < _E, gates, 0.0)
    idx_ref[...] = jnp.where(lane == 0, i1, jnp.where(lane == 1, i2, 0))
    g_ref[...] = jnp.where(lane == 0, g1, jnp.where(lane == 1, g2, 0.0))
    emask = (lane[:1] < _E).astype(jnp.float32)
    imp = jnp.sum(gates, axis=0, keepdims=True)
    load = jnp.sum((gates > 0).astype(jnp.float32), axis=0, keepdims=True)

    def cv2(v):
        m = jnp.sum(v * emask) / _E
        var = jnp.sum((v - m) ** 2 * emask) / (_E - 1)
        return var / (m * m + 1e-10)

    loss = (cv2(imp) + cv2(load)) * 0.01
    loss_ref[...] = jnp.full((8, 128), loss, dtype=jnp.float32)


def _gating_call(x, wg_pad, interpret=False):
    return pl.pallas_call(
        _gating_body,
        out_shape=(
            jax.ShapeDtypeStruct((_N, _EP), jnp.int32),
            jax.ShapeDtypeStruct((_N, _EP), jnp.float32),
            jax.ShapeDtypeStruct((8, 128), jnp.float32),
        ),
        interpret=interpret,
    )(x, wg_pad)


def _gmm_body(meta_ref, xs_ref, w1_ref, b1_ref, w2_ref, b2_ref, gp_ref, out_ref):
    t = pl.program_id(0)

    @pl.when(meta_ref[2, t] == 1)
    def _():
        xk = xs_ref[...].astype(jnp.bfloat16)
        h = jnp.dot(xk, w1_ref[0], preferred_element_type=jnp.float32)
        h = jnp.maximum(h + b1_ref[0], 0.0).astype(jnp.bfloat16)
        o = jnp.dot(h, w2_ref[0], preferred_element_type=jnp.float32)
        o = o + b2_ref[0]
        out_ref[...] = o * gp_ref[0, 0][:, None]


def _gmm_call(meta, xs, w1, b1, w2, b2, gp3, interpret=False):
    grid_spec = pltpu.PrefetchScalarGridSpec(
        num_scalar_prefetch=1,
        grid=(_NT,),
        in_specs=[
            pl.BlockSpec((_T, _D), lambda t, m: (m[1, t], 0)),
            pl.BlockSpec((1, _D, _H), lambda t, m: (m[0, t], 0, 0)),
            pl.BlockSpec((1, 1, _H), lambda t, m: (m[0, t], 0, 0)),
            pl.BlockSpec((1, _H, _O), lambda t, m: (m[0, t], 0, 0)),
            pl.BlockSpec((1, 1, _O), lambda t, m: (m[0, t], 0, 0)),
            pl.BlockSpec((1, 1, _T), lambda t, m: (m[1, t], 0, 0)),
        ],
        out_specs=pl.BlockSpec((_T, _O), lambda t, m: (m[1, t], 0)),
    )
    return pl.pallas_call(
        _gmm_body,
        grid_spec=grid_spec,
        out_shape=jax.ShapeDtypeStruct((_PAD, _O), jnp.float32),
        interpret=interpret,
    )(meta, xs, w1, b1, w2, b2, gp3)


@functools.lru_cache(maxsize=None)
def _sc_kernels():
    mesh = plsc.VectorSubcoreMesh(core_axis_name="c", subcore_axis_name="s")

    nch = _DROWS // _DCH
    nstr = _DCH // _DSR

    @functools.partial(
        pl.kernel,
        out_type=jax.ShapeDtypeStruct((_PAD, _D), jnp.float32),
        mesh=mesh,
        scratch_types=(
            [pltpu.VMEM((_DROWS,), jnp.int32)]
            + [pltpu.VMEM((_DSR, _D), jnp.float32) for _ in range(2 * nstr)]
            + [pltpu.SemaphoreType.DMA for _ in range(4 * nstr)]
        ),
    )
    def _sc_dispatch(x_hbm, tok_hbm, out_hbm, idx_v, *rest):
        bufs = (rest[:nstr], rest[nstr:2 * nstr])
        gsem = (rest[2 * nstr:3 * nstr], rest[3 * nstr:4 * nstr])
        wsem = (rest[4 * nstr:5 * nstr], rest[5 * nstr:6 * nstr])
        wid = lax.axis_index("s") * _NC + lax.axis_index("c")
        base = wid * _DROWS
        pltpu.sync_copy(tok_hbm.at[pl.ds(base, _DROWS)], idx_v)

        def start(c):
            s = c % 2
            return [
                pltpu.async_copy(
                    x_hbm.at[idx_v.at[pl.ds(c * _DCH + k * _DSR, _DSR)]],
                    bufs[s][k], gsem[s][k])
                for k in range(nstr)
            ]

        gathers = [None] * nch
        writes = [None] * nch
        gathers[0] = start(0)
        for c in range(nch):
            s = c % 2
            for g in gathers[c]:
                g.wait()
            if c + 1 < nch:
                if c >= 1:
                    for w in writes[c - 1]:
                        w.wait()
                gathers[c + 1] = start(c + 1)
            writes[c] = [
                pltpu.async_copy(
                    bufs[s][k],
                    out_hbm.at[pl.ds(base + c * _DCH + k * _DSR, _DSR)],
                    wsem[s][k])
                for k in range(nstr)
            ]
        if nch >= 2:
            for w in writes[nch - 2]:
                w.wait()
        for w in writes[nch - 1]:
            w.wait()

    @functools.partial(
        pl.kernel,
        out_type=jax.ShapeDtypeStruct((_N, _O), jnp.float32),
        mesh=mesh,
        scratch_types=[
            pltpu.VMEM((_TOKW,), jnp.int32),
            pltpu.VMEM((_TOKW,), jnp.int32),
            pltpu.VMEM((_CCH, _O), jnp.float32),
            pltpu.VMEM((_CCH, _O), jnp.float32),
            pltpu.VMEM((_CCH, _O), jnp.float32),
            pltpu.VMEM((_CCH, _O), jnp.float32),
            pltpu.SemaphoreType.DMA,
            pltpu.SemaphoreType.DMA,
            pltpu.SemaphoreType.DMA,
            pltpu.SemaphoreType.DMA,
            pltpu.SemaphoreType.DMA,
            pltpu.SemaphoreType.DMA,
        ],
    )
    def _sc_combine(rows_hbm, pos0_hbm, pos1_hbm, y_hbm,
                    i0_v, i1_v, a0, a1, c0, c1,
                    ga0, ga1, gb0, gb1, wa, wb):
        wid = lax.axis_index("s") * _NC + lax.axis_index("c")
        base = wid * _TOKW
        pltpu.sync_copy(pos0_hbm.at[pl.ds(base, _TOKW)], i0_v)
        pltpu.sync_copy(pos1_hbm.at[pl.ds(base, _TOKW)], i1_v)
        slots = ((a0, a1, ga0, ga1, wa), (c0, c1, gb0, gb1, wb))
        nch = _TOKW // _CCH
        gathers = [None] * nch
        writes = [None] * nch

        def start(c):
            r0, r1, gs0, gs1, _ = slots[c % 2]
            sl = pl.ds(c * _CCH, _CCH)
            cp0 = pltpu.async_copy(rows_hbm.at[i0_v.at[sl]], r0, gs0)
            cp1 = pltpu.async_copy(rows_hbm.at[i1_v.at[sl]], r1, gs1)
            return (cp0, cp1)

        gathers[0] = start(0)
        for c in range(nch):
            r0, r1, _, _, ws = slots[c % 2]
            gathers[c][0].wait()
            gathers[c][1].wait()
            if c + 1 < nch:
                if c >= 1:
                    writes[c - 1].wait()
                gathers[c + 1] = start(c + 1)

            def row_body(r, _):
                def vec_body(v, _):
                    sl = pl.ds(v * 16, 16)
                    r0[r, sl] = r0[r, sl] + r1[r, sl]
                    return 0

                return lax.fori_loop(0, _O // 16, vec_body, 0)

            lax.fori_loop(0, _CCH, row_body, 0)
            writes[c] = pltpu.async_copy(
                r0, y_hbm.at[pl.ds(base + c * _CCH, _CCH)], ws)
        if nch >= 2:
            writes[nch - 2].wait()
        writes[nch - 1].wait()

    return _sc_dispatch, _sc_combine


def _routing(i12, g12):
    e_flat = i12.reshape(-1)
    g_flat = g12.reshape(-1)
    order = jnp.argsort(e_flat, stable=True)
    e_sorted = e_flat[order]
    counts = jnp.bincount(e_flat, length=_E).astype(jnp.int32)
    tiles = (counts + _T - 1) // _T
    zero1 = jnp.zeros((1,), jnp.int32)
    pad_start = jnp.concatenate([zero1, jnp.cumsum(tiles * _T)])
    start = jnp.concatenate([zero1, jnp.cumsum(counts)])
    j = jnp.arange(_S, dtype=jnp.int32)
    dest = pad_start[e_sorted] + (j - start[e_sorted])
    tok_flat = jnp.arange(_S, dtype=jnp.int32) // _K
    tok_pad = jnp.zeros((_PAD,), jnp.int32).at[dest].set(tok_flat[order])
    g_pad = jnp.zeros((_PAD,), jnp.float32).at[dest].set(g_flat[order])
    pos_flat = jnp.zeros((_S,), jnp.int32).at[order].set(dest)
    pos = pos_flat.reshape(_N, _K)
    nv = pad_start[_E] // _T
    tvec = jnp.arange(_NT, dtype=jnp.int32)
    eid = jnp.clip(jnp.searchsorted(pad_start[1:], tvec * _T, side="right"),
                   0, _E - 1).astype(jnp.int32)
    eid = jnp.where(tvec < nv, eid, eid[nv - 1])
    blk = jnp.where(tvec < nv, tvec, nv - 1)
    valid = (tvec < nv).astype(jnp.int32)
    meta = jnp.stack([eid, blk, valid])
    return tok_pad, g_pad, pos, meta


def kernel(x, w_gate, W1, b1, W2, b2):
    wg_pad = jnp.zeros((_D, _EP), jnp.float32).at[:, :_E].set(w_gate)
    idx8, g8, loss8 = _gating_call(x, wg_pad)
    i12 = idx8[:, :_K]
    g12 = g8[:, :_K]
    tok_pad, g_pad, pos, meta = _routing(i12, g12)
    sc_dispatch, sc_combine = _sc_kernels()
    xs = sc_dispatch(x, tok_pad)
    gp3 = g_pad.reshape(_NT, 1, _T)
    rows = _gmm_call(meta, xs, W1.astype(jnp.bfloat16), b1.reshape(_E, 1, _H),
                     W2.astype(jnp.bfloat16), b2.reshape(_E, 1, _O), gp3)
    y = sc_combine(rows, pos[:, 0], pos[:, 1])
    return y, loss8[0, 0]


# speedup vs baseline: 2.2928x; 1.5474x over previous
import functools

import jax
import jax.numpy as jnp
from jax import lax
from jax.experimental import pallas as pl
from jax.experimental.pallas import tpu as pltpu
from jax.experimental.pallas import tpu_sc as plsc

_N, _D, _H, _O, _E, _K = 2048, 1024, 1024, 1024, 8, 2
_S = _N * _K
_T = 256
_NT = (_S + _E * _T) // _T
_PAD = _NT * _T
_EP = 128

_NC, _NS = 2, 16
_NW = _NC * _NS
_DROWS = _PAD // _NW
_DCH = 48
_DSR = 16
_TOKW = _N // _NW
_CCH = 16


def _gating_body(x_ref, wg_ref, idx_ref, g_ref, loss_ref, xi_ref):
    x = x_ref[...]
    xb = x.astype(jnp.bfloat16)
    lo = lax.convert_element_type(
        lax.bitcast_convert_type(xb[:, : _D // 2], jnp.uint16), jnp.uint32)
    hi = lax.convert_element_type(
        lax.bitcast_convert_type(xb[:, _D // 2:], jnp.uint16), jnp.uint32)
    xi_ref[...] = lax.bitcast_convert_type((hi << 16) | lo, jnp.int32)
    wg = wg_ref[...]
    logits = jnp.dot(x, wg, preferred_element_type=jnp.float32)
    lane = lax.broadcasted_iota(jnp.int32, (_N, _EP), 1)
    neg = jnp.float32(-jnp.inf)
    logits = jnp.where(lane < _E, logits, neg)
    m1 = jnp.max(logits, axis=1, keepdims=True)
    i1 = jnp.min(jnp.where(logits == m1, lane, _EP), axis=1, keepdims=True)
    masked = jnp.where(lane == i1, neg, logits)
    m2 = jnp.max(masked, axis=1, keepdims=True)
    i2 = jnp.min(jnp.where(masked == m2, lane, _EP), axis=1, keepdims=True)
    t = jnp.exp(m2 - m1)
    g1 = 1.0 / (1.0 + t)
    g2 = t / (1.0 + t)
    gates = jnp.where(lane == i1, g1, 0.0) + jnp.where(lane == i2, g2, 0.0)
    gates = jnp.where(lane < _E, gates, 0.0)
    idx_ref[...] = jnp.where(lane == 0, i1, jnp.where(lane == 1, i2, 0))
    g_ref[...] = jnp.where(lane == 0, g1, jnp.where(lane == 1, g2, 0.0))
    emask = (lane[:1] < _E).astype(jnp.float32)
    imp = jnp.sum(gates, axis=0, keepdims=True)
    load = jnp.sum((gates > 0).astype(jnp.float32), axis=0, keepdims=True)

    def cv2(v):
        m = jnp.sum(v * emask) / _E
        var = jnp.sum((v - m) ** 2 * emask) / (_E - 1)
        return var / (m * m + 1e-10)

    loss = (cv2(imp) + cv2(load)) * 0.01
    loss_ref[...] = jnp.full((8, 128), loss, dtype=jnp.float32)


def _gating_call(x, wg_pad, interpret=False):
    return pl.pallas_call(
        _gating_body,
        out_shape=(
            jax.ShapeDtypeStruct((_N, _EP), jnp.int32),
            jax.ShapeDtypeStruct((_N, _EP), jnp.float32),
            jax.ShapeDtypeStruct((8, 128), jnp.float32),
            jax.ShapeDtypeStruct((_N, _D // 2), jnp.int32),
        ),
        interpret=interpret,
    )(x, wg_pad)


def _gmm_body(meta_ref, xs_ref, w1_ref, b1_ref, w2_ref, b2_ref, gp_ref, out_ref):
    t = pl.program_id(0)

    @pl.when(meta_ref[2, t] == 1)
    def _():
        v = lax.bitcast_convert_type(xs_ref[...], jnp.uint32)
        lo = lax.bitcast_convert_type(
            lax.convert_element_type(v & 0xFFFF, jnp.uint16), jnp.bfloat16)
        hi = lax.bitcast_convert_type(
            lax.convert_element_type(v >> 16, jnp.uint16), jnp.bfloat16)
        xk = jnp.concatenate([lo, hi], axis=1)
        h = jnp.dot(xk, w1_ref[0], preferred_element_type=jnp.float32)
        h = jnp.maximum(h + b1_ref[0], 0.0).astype(jnp.bfloat16)
        o = jnp.dot(h, w2_ref[0], preferred_element_type=jnp.float32)
        o = o + b2_ref[0]
        out_ref[...] = o * gp_ref[0, 0][:, None]


def _gmm_call(meta, xs, w1, b1, w2, b2, gp3, interpret=False):
    grid_spec = pltpu.PrefetchScalarGridSpec(
        num_scalar_prefetch=1,
        grid=(_NT,),
        in_specs=[
            pl.BlockSpec((_T, _D // 2), lambda t, m: (m[1, t], 0)),
            pl.BlockSpec((1, _D, _H), lambda t, m: (m[0, t], 0, 0)),
            pl.BlockSpec((1, 1, _H), lambda t, m: (m[0, t], 0, 0)),
            pl.BlockSpec((1, _H, _O), lambda t, m: (m[0, t], 0, 0)),
            pl.BlockSpec((1, 1, _O), lambda t, m: (m[0, t], 0, 0)),
            pl.BlockSpec((1, 1, _T), lambda t, m: (m[1, t], 0, 0)),
        ],
        out_specs=pl.BlockSpec((_T, _O), lambda t, m: (m[1, t], 0)),
    )
    return pl.pallas_call(
        _gmm_body,
        grid_spec=grid_spec,
        out_shape=jax.ShapeDtypeStruct((_PAD, _O), jnp.float32),
        interpret=interpret,
    )(meta, xs, w1, b1, w2, b2, gp3)


@functools.lru_cache(maxsize=None)
def _sc_kernels():
    mesh = plsc.VectorSubcoreMesh(core_axis_name="c", subcore_axis_name="s")

    @functools.partial(
        pl.kernel,
        out_type=jax.ShapeDtypeStruct((_PAD, _D // 2), jnp.int32),
        mesh=mesh,
        scratch_types=[
            pltpu.VMEM((_TOKW, _D // 2), jnp.int32),
            pltpu.VMEM((_K, _TOKW), jnp.int32),
            pltpu.SemaphoreType.DMA,
            pltpu.SemaphoreType.DMA,
        ],
    )
    def _sc_dispatch(xi_hbm, posw_hbm, out_hbm, xloc, idxs, s0, s1):
        wid = lax.axis_index("s") * _NC + lax.axis_index("c")
        pltpu.sync_copy(xi_hbm.at[pl.ds(wid * _TOKW, _TOKW)], xloc)
        pltpu.sync_copy(posw_hbm.at[wid], idxs)
        c0 = pltpu.async_copy(xloc, out_hbm.at[idxs.at[0]], s0)
        c1 = pltpu.async_copy(xloc, out_hbm.at[idxs.at[1]], s1)
        c0.wait()
        c1.wait()

    @functools.partial(
        pl.kernel,
        out_type=jax.ShapeDtypeStruct((_N, _O), jnp.float32),
        mesh=mesh,
        scratch_types=[
            pltpu.VMEM((_TOKW,), jnp.int32),
            pltpu.VMEM((_TOKW,), jnp.int32),
            pltpu.VMEM((_CCH, _O), jnp.float32),
            pltpu.VMEM((_CCH, _O), jnp.float32),
            pltpu.VMEM((_CCH, _O), jnp.float32),
            pltpu.VMEM((_CCH, _O), jnp.float32),
            pltpu.SemaphoreType.DMA,
            pltpu.SemaphoreType.DMA,
            pltpu.SemaphoreType.DMA,
            pltpu.SemaphoreType.DMA,
            pltpu.SemaphoreType.DMA,
            pltpu.SemaphoreType.DMA,
        ],
    )
    def _sc_combine(rows_hbm, pos0_hbm, pos1_hbm, y_hbm,
                    i0_v, i1_v, a0, a1, c0, c1,
                    ga0, ga1, gb0, gb1, wa, wb):
        wid = lax.axis_index("s") * _NC + lax.axis_index("c")
        base = wid * _TOKW
        pltpu.sync_copy(pos0_hbm.at[pl.ds(base, _TOKW)], i0_v)
        pltpu.sync_copy(pos1_hbm.at[pl.ds(base, _TOKW)], i1_v)
        slots = ((a0, a1, ga0, ga1, wa), (c0, c1, gb0, gb1, wb))
        nch = _TOKW // _CCH
        gathers = [None] * nch
        writes = [None] * nch

        def start(c):
            r0, r1, gs0, gs1, _ = slots[c % 2]
            sl = pl.ds(c * _CCH, _CCH)
            cp0 = pltpu.async_copy(rows_hbm.at[i0_v.at[sl]], r0, gs0)
            cp1 = pltpu.async_copy(rows_hbm.at[i1_v.at[sl]], r1, gs1)
            return (cp0, cp1)

        gathers[0] = start(0)
        for c in range(nch):
            r0, r1, _, _, ws = slots[c % 2]
            gathers[c][0].wait()
            gathers[c][1].wait()
            if c + 1 < nch:
                if c >= 1:
                    writes[c - 1].wait()
                gathers[c + 1] = start(c + 1)

            def row_body(r, _):
                def vec_body(v, _):
                    sl = pl.ds(v * 16, 16)
                    r0[r, sl] = r0[r, sl] + r1[r, sl]
                    return 0

                return lax.fori_loop(0, _O // 16, vec_body, 0)

            lax.fori_loop(0, _CCH, row_body, 0)
            writes[c] = pltpu.async_copy(
                r0, y_hbm.at[pl.ds(base + c * _CCH, _CCH)], ws)
        if nch >= 2:
            writes[nch - 2].wait()
        writes[nch - 1].wait()

    return _sc_dispatch, _sc_combine


def _routing(i12, g12):
    e_flat = i12.reshape(-1)
    g_flat = g12.reshape(-1)
    order = jnp.argsort(e_flat, stable=True)
    e_sorted = e_flat[order]
    counts = jnp.bincount(e_flat, length=_E).astype(jnp.int32)
    tiles = (counts + _T - 1) // _T
    zero1 = jnp.zeros((1,), jnp.int32)
    pad_start = jnp.concatenate([zero1, jnp.cumsum(tiles * _T)])
    start = jnp.concatenate([zero1, jnp.cumsum(counts)])
    j = jnp.arange(_S, dtype=jnp.int32)
    dest = pad_start[e_sorted] + (j - start[e_sorted])
    g_pad = jnp.zeros((_PAD,), jnp.float32).at[dest].set(g_flat[order])
    pos_flat = jnp.zeros((_S,), jnp.int32).at[order].set(dest)
    pos = pos_flat.reshape(_N, _K)
    nv = pad_start[_E] // _T
    tvec = jnp.arange(_NT, dtype=jnp.int32)
    eid = jnp.clip(jnp.searchsorted(pad_start[1:], tvec * _T, side="right"),
                   0, _E - 1).astype(jnp.int32)
    eid = jnp.where(tvec < nv, eid, eid[nv - 1])
    blk = jnp.where(tvec < nv, tvec, nv - 1)
    valid = (tvec < nv).astype(jnp.int32)
    meta = jnp.stack([eid, blk, valid])
    return g_pad, pos, meta


def kernel(x, w_gate, W1, b1, W2, b2):
    wg_pad = jnp.zeros((_D, _EP), jnp.float32).at[:, :_E].set(w_gate)
    idx8, g8, loss8, xi32 = _gating_call(x, wg_pad)
    i12 = idx8[:, :_K]
    g12 = g8[:, :_K]
    g_pad, pos, meta = _routing(i12, g12)
    sc_dispatch, sc_combine = _sc_kernels()
    posw = pos.reshape(_NW, _TOKW, _K).transpose(0, 2, 1)
    xs = sc_dispatch(xi32, posw)
    gp3 = g_pad.reshape(_NT, 1, _T)
    rows = _gmm_call(meta, xs, W1.astype(jnp.bfloat16), b1.reshape(_E, 1, _H),
                     W2.astype(jnp.bfloat16), b2.reshape(_E, 1, _O), gp3)
    y = sc_combine(rows, pos[:, 0], pos[:, 1])
    return y, loss8[0, 0]


# speedup vs baseline: 2.7518x; 1.2002x over previous
import functools

import jax
import jax.numpy as jnp
from jax import lax
from jax.experimental import pallas as pl
from jax.experimental.pallas import tpu as pltpu
from jax.experimental.pallas import tpu_sc as plsc

_N, _D, _H, _O, _E, _K = 2048, 1024, 1024, 1024, 8, 2
_S = _N * _K
_T = 256
_NT = (_S + _E * _T) // _T
_PAD = _NT * _T
_EP = 128

_NC, _NS = 2, 16
_NW = _NC * _NS
_DROWS = _PAD // _NW
_DCH = 48
_DSR = 16
_TOKW = _N // _NW
_CCH = 16


def _gating_body(x_ref, wg_ref, idx_ref, g_ref, loss_ref, xi_ref):
    x = x_ref[...]
    xb = x.astype(jnp.bfloat16)
    lo = lax.convert_element_type(
        lax.bitcast_convert_type(xb[:, : _D // 2], jnp.uint16), jnp.uint32)
    hi = lax.convert_element_type(
        lax.bitcast_convert_type(xb[:, _D // 2:], jnp.uint16), jnp.uint32)
    xi_ref[...] = lax.bitcast_convert_type((hi << 16) | lo, jnp.int32)
    wg = wg_ref[...]
    logits = jnp.dot(x, wg, preferred_element_type=jnp.float32)
    lane = lax.broadcasted_iota(jnp.int32, (_N, _EP), 1)
    neg = jnp.float32(-jnp.inf)
    logits = jnp.where(lane < _E, logits, neg)
    m1 = jnp.max(logits, axis=1, keepdims=True)
    i1 = jnp.min(jnp.where(logits == m1, lane, _EP), axis=1, keepdims=True)
    masked = jnp.where(lane == i1, neg, logits)
    m2 = jnp.max(masked, axis=1, keepdims=True)
    i2 = jnp.min(jnp.where(masked == m2, lane, _EP), axis=1, keepdims=True)
    t = jnp.exp(m2 - m1)
    g1 = 1.0 / (1.0 + t)
    g2 = t / (1.0 + t)
    gates = jnp.where(lane == i1, g1, 0.0) + jnp.where(lane == i2, g2, 0.0)
    gates = jnp.where(lane < _E, gates, 0.0)
    idx_ref[...] = jnp.where(lane == 0, i1, jnp.where(lane == 1, i2, 0))
    g_ref[...] = jnp.where(lane == 0, g1, jnp.where(lane == 1, g2, 0.0))
    emask = (lane[:1] < _E).astype(jnp.float32)
    imp = jnp.sum(gates, axis=0, keepdims=True)
    load = jnp.sum((gates > 0).astype(jnp.float32), axis=0, keepdims=True)

    def cv2(v):
        m = jnp.sum(v * emask) / _E
        var = jnp.sum((v - m) ** 2 * emask) / (_E - 1)
        return var / (m * m + 1e-10)

    loss = (cv2(imp) + cv2(load)) * 0.01
    loss_ref[...] = jnp.full((8, 128), loss, dtype=jnp.float32)


def _gating_call(x, wg_pad, interpret=False):
    return pl.pallas_call(
        _gating_body,
        out_shape=(
            jax.ShapeDtypeStruct((_N, _EP), jnp.int32),
            jax.ShapeDtypeStruct((_N, _EP), jnp.float32),
            jax.ShapeDtypeStruct((8, 128), jnp.float32),
            jax.ShapeDtypeStruct((_N, _D // 2), jnp.int32),
        ),
        interpret=interpret,
    )(x, wg_pad)


def _gmm_body(meta_ref, xs_ref, w1_ref, b1_ref, w2_ref, b2_ref, gp_ref, out_ref):
    t = pl.program_id(0)

    @pl.when(meta_ref[2, t] == 1)
    def _():
        v = lax.bitcast_convert_type(xs_ref[...], jnp.uint32)
        lo = lax.bitcast_convert_type(
            lax.convert_element_type(v & 0xFFFF, jnp.uint16), jnp.bfloat16)
        hi = lax.bitcast_convert_type(
            lax.convert_element_type(v >> 16, jnp.uint16), jnp.bfloat16)
        xk = jnp.concatenate([lo, hi], axis=1)
        h = jnp.dot(xk, w1_ref[0], preferred_element_type=jnp.float32)
        h = jnp.maximum(h + b1_ref[0], 0.0).astype(jnp.bfloat16)
        o = jnp.dot(h, w2_ref[0], preferred_element_type=jnp.float32)
        o = o + b2_ref[0]
        out_ref[...] = o * gp_ref[0, 0][:, None]


def _gmm_call(meta, xs, w1, b1, w2, b2, gp3, interpret=False):
    grid_spec = pltpu.PrefetchScalarGridSpec(
        num_scalar_prefetch=1,
        grid=(_NT,),
        in_specs=[
            pl.BlockSpec((_T, _D // 2), lambda t, m: (m[1, t], 0)),
            pl.BlockSpec((1, _D, _H), lambda t, m: (m[0, t], 0, 0)),
            pl.BlockSpec((1, 1, _H), lambda t, m: (m[0, t], 0, 0)),
            pl.BlockSpec((1, _H, _O), lambda t, m: (m[0, t], 0, 0)),
            pl.BlockSpec((1, 1, _O), lambda t, m: (m[0, t], 0, 0)),
            pl.BlockSpec((1, 1, _T), lambda t, m: (m[1, t], 0, 0)),
        ],
        out_specs=pl.BlockSpec((_T, _O), lambda t, m: (m[1, t], 0)),
    )
    return pl.pallas_call(
        _gmm_body,
        grid_spec=grid_spec,
        out_shape=jax.ShapeDtypeStruct((_PAD, _O), jnp.float32),
        interpret=interpret,
    )(meta, xs, w1, b1, w2, b2, gp3)


@functools.lru_cache(maxsize=None)
def _sc_kernels():
    mesh = plsc.VectorSubcoreMesh(core_axis_name="c", subcore_axis_name="s")

    @functools.partial(
        pl.kernel,
        out_type=jax.ShapeDtypeStruct((_PAD, _D // 2), jnp.int32),
        mesh=mesh,
        scratch_types=[
            pltpu.VMEM((_TOKW, _D // 2), jnp.int32),
            pltpu.VMEM((_K, _TOKW), jnp.int32),
            pltpu.SemaphoreType.DMA,
            pltpu.SemaphoreType.DMA,
        ],
    )
    def _sc_dispatch(xi_hbm, posw_hbm, out_hbm, xloc, idxs, s0, s1):
        wid = lax.axis_index("s") * _NC + lax.axis_index("c")
        pltpu.sync_copy(xi_hbm.at[pl.ds(wid * _TOKW, _TOKW)], xloc)
        pltpu.sync_copy(posw_hbm.at[wid], idxs)
        c0 = pltpu.async_copy(xloc, out_hbm.at[idxs.at[0]], s0)
        c1 = pltpu.async_copy(xloc, out_hbm.at[idxs.at[1]], s1)
        c0.wait()
        c1.wait()

    @functools.partial(
        pl.kernel,
        out_type=jax.ShapeDtypeStruct((_N, _O), jnp.float32),
        mesh=mesh,
        scratch_types=[
            pltpu.VMEM((_TOKW,), jnp.int32),
            pltpu.VMEM((_TOKW,), jnp.int32),
            pltpu.VMEM((_CCH, _O), jnp.float32),
            pltpu.VMEM((_CCH, _O), jnp.float32),
            pltpu.VMEM((_CCH, _O), jnp.float32),
            pltpu.VMEM((_CCH, _O), jnp.float32),
            pltpu.SemaphoreType.DMA,
            pltpu.SemaphoreType.DMA,
            pltpu.SemaphoreType.DMA,
            pltpu.SemaphoreType.DMA,
            pltpu.SemaphoreType.DMA,
            pltpu.SemaphoreType.DMA,
        ],
    )
    def _sc_combine(rows_hbm, pos0_hbm, pos1_hbm, y_hbm,
                    i0_v, i1_v, a0, a1, c0, c1,
                    ga0, ga1, gb0, gb1, wa, wb):
        wid = lax.axis_index("s") * _NC + lax.axis_index("c")
        base = wid * _TOKW
        pltpu.sync_copy(pos0_hbm.at[pl.ds(base, _TOKW)], i0_v)
        pltpu.sync_copy(pos1_hbm.at[pl.ds(base, _TOKW)], i1_v)
        slots = ((a0, a1, ga0, ga1, wa), (c0, c1, gb0, gb1, wb))
        nch = _TOKW // _CCH
        gathers = [None] * nch
        writes = [None] * nch

        def start(c):
            r0, r1, gs0, gs1, _ = slots[c % 2]
            sl = pl.ds(c * _CCH, _CCH)
            cp0 = pltpu.async_copy(rows_hbm.at[i0_v.at[sl]], r0, gs0)
            cp1 = pltpu.async_copy(rows_hbm.at[i1_v.at[sl]], r1, gs1)
            return (cp0, cp1)

        gathers[0] = start(0)
        for c in range(nch):
            r0, r1, _, _, ws = slots[c % 2]
            gathers[c][0].wait()
            gathers[c][1].wait()
            if c + 1 < nch:
                if c >= 1:
                    writes[c - 1].wait()
                gathers[c + 1] = start(c + 1)

            def row_body(r, _):
                def vec_body(v, _):
                    sl = pl.ds(v * 16, 16)
                    r0[r, sl] = r0[r, sl] + r1[r, sl]
                    return 0

                return lax.fori_loop(0, _O // 16, vec_body, 0)

            lax.fori_loop(0, _CCH, row_body, 0)
            writes[c] = pltpu.async_copy(
                r0, y_hbm.at[pl.ds(base + c * _CCH, _CCH)], ws)
        if nch >= 2:
            writes[nch - 2].wait()
        writes[nch - 1].wait()

    return _sc_dispatch, _sc_combine


def _routing(i12, g12):
    e_flat = i12.reshape(-1)
    g_flat = g12.reshape(-1)
    oh = (e_flat[:, None] == jnp.arange(_E, dtype=jnp.int32)[None, :])
    oh = oh.astype(jnp.int32)
    csum = jnp.cumsum(oh, axis=0)
    counts = csum[-1]
    rank = jnp.sum(csum * oh, axis=1) - 1
    tiles = (counts + _T - 1) // _T
    zero1 = jnp.zeros((1,), jnp.int32)
    pad_start = jnp.concatenate([zero1, jnp.cumsum(tiles * _T)])
    dest = pad_start[e_flat] + rank
    g_pad = jnp.zeros((_PAD,), jnp.float32).at[dest].set(g_flat)
    pos = dest.reshape(_N, _K)
    nv = pad_start[_E] // _T
    tvec = jnp.arange(_NT, dtype=jnp.int32)
    eid = jnp.clip(jnp.searchsorted(pad_start[1:], tvec * _T, side="right"),
                   0, _E - 1).astype(jnp.int32)
    eid = jnp.where(tvec < nv, eid, eid[nv - 1])
    blk = jnp.where(tvec < nv, tvec, nv - 1)
    valid = (tvec < nv).astype(jnp.int32)
    meta = jnp.stack([eid, blk, valid])
    return g_pad, pos, meta


def kernel(x, w_gate, W1, b1, W2, b2):
    wg_pad = jnp.zeros((_D, _EP), jnp.float32).at[:, :_E].set(w_gate)
    idx8, g8, loss8, xi32 = _gating_call(x, wg_pad)
    i12 = idx8[:, :_K]
    g12 = g8[:, :_K]
    g_pad, pos, meta = _routing(i12, g12)
    sc_dispatch, sc_combine = _sc_kernels()
    posw = pos.reshape(_NW, _TOKW, _K).transpose(0, 2, 1)
    xs = sc_dispatch(xi32, posw)
    gp3 = g_pad.reshape(_NT, 1, _T)
    rows = _gmm_call(meta, xs, W1.astype(jnp.bfloat16), b1.reshape(_E, 1, _H),
                     W2.astype(jnp.bfloat16), b2.reshape(_E, 1, _O), gp3)
    y = sc_combine(rows, pos[:, 0], pos[:, 1])
    return y, loss8[0, 0]


# speedup vs baseline: 2.8269x; 1.0273x over previous
import functools

import jax
import jax.numpy as jnp
from jax import lax
from jax.experimental import pallas as pl
from jax.experimental.pallas import tpu as pltpu
from jax.experimental.pallas import tpu_sc as plsc

_N, _D, _H, _O, _E, _K = 2048, 1024, 1024, 1024, 8, 2
_S = _N * _K
_T = 256
_NT = (_S + _E * _T) // _T
_PAD = _NT * _T
_EP = 128

_NC, _NS = 2, 16
_NW = _NC * _NS
_DROWS = _PAD // _NW
_DCH = 48
_DSR = 16
_TOKW = _N // _NW
_CCH = 16


def _gating_body(x_ref, wg_ref, idx_ref, g_ref, loss_ref, xi_ref):
    x = x_ref[...]
    xb = x.astype(jnp.bfloat16)
    lo = lax.convert_element_type(
        lax.bitcast_convert_type(xb[:, : _D // 2], jnp.uint16), jnp.uint32)
    hi = lax.convert_element_type(
        lax.bitcast_convert_type(xb[:, _D // 2:], jnp.uint16), jnp.uint32)
    xi_ref[...] = lax.bitcast_convert_type((hi << 16) | lo, jnp.int32)
    wg = wg_ref[...]
    logits = jnp.dot(x, wg, preferred_element_type=jnp.float32)
    lane = lax.broadcasted_iota(jnp.int32, (_N, _EP), 1)
    neg = jnp.float32(-jnp.inf)
    logits = jnp.where(lane < _E, logits, neg)
    m1 = jnp.max(logits, axis=1, keepdims=True)
    i1 = jnp.min(jnp.where(logits == m1, lane, _EP), axis=1, keepdims=True)
    masked = jnp.where(lane == i1, neg, logits)
    m2 = jnp.max(masked, axis=1, keepdims=True)
    i2 = jnp.min(jnp.where(masked == m2, lane, _EP), axis=1, keepdims=True)
    t = jnp.exp(m2 - m1)
    g1 = 1.0 / (1.0 + t)
    g2 = t / (1.0 + t)
    gates = jnp.where(lane == i1, g1, 0.0) + jnp.where(lane == i2, g2, 0.0)
    gates = jnp.where(lane < _E, gates, 0.0)
    idx_ref[...] = jnp.where(lane == 0, i1, jnp.where(lane == 1, i2, 0))
    g_ref[...] = jnp.where(lane == 0, g1, jnp.where(lane == 1, g2, 0.0))
    emask = (lane[:1] < _E).astype(jnp.float32)
    imp = jnp.sum(gates, axis=0, keepdims=True)
    load = jnp.sum((gates > 0).astype(jnp.float32), axis=0, keepdims=True)

    def cv2(v):
        m = jnp.sum(v * emask) / _E
        var = jnp.sum((v - m) ** 2 * emask) / (_E - 1)
        return var / (m * m + 1e-10)

    loss = (cv2(imp) + cv2(load)) * 0.01
    loss_ref[...] = jnp.full((8, 128), loss, dtype=jnp.float32)


def _gating_call(x, wg_pad, interpret=False):
    return pl.pallas_call(
        _gating_body,
        out_shape=(
            jax.ShapeDtypeStruct((_N, _EP), jnp.int32),
            jax.ShapeDtypeStruct((_N, _EP), jnp.float32),
            jax.ShapeDtypeStruct((8, 128), jnp.float32),
            jax.ShapeDtypeStruct((_N, _D // 2), jnp.int32),
        ),
        interpret=interpret,
    )(x, wg_pad)


def _gmm_body(meta_ref, xs_ref, w1_ref, b1_ref, w2_ref, b2_ref, out_ref):
    t = pl.program_id(0)

    @pl.when(meta_ref[2, t] == 1)
    def _():
        v = lax.bitcast_convert_type(xs_ref[...], jnp.uint32)
        lo = lax.bitcast_convert_type(
            lax.convert_element_type(v & 0xFFFF, jnp.uint16), jnp.bfloat16)
        hi = lax.bitcast_convert_type(
            lax.convert_element_type(v >> 16, jnp.uint16), jnp.bfloat16)
        xk = jnp.concatenate([lo, hi], axis=1)
        h = jnp.dot(xk, w1_ref[0], preferred_element_type=jnp.float32)
        h = jnp.maximum(h + b1_ref[0], 0.0).astype(jnp.bfloat16)
        o = jnp.dot(h, w2_ref[0], preferred_element_type=jnp.float32)
        out_ref[...] = o + b2_ref[0]


def _gmm_call(meta, xs, w1, b1, w2, b2, interpret=False):
    grid_spec = pltpu.PrefetchScalarGridSpec(
        num_scalar_prefetch=1,
        grid=(_NT,),
        in_specs=[
            pl.BlockSpec((_T, _D // 2), lambda t, m: (m[1, t], 0)),
            pl.BlockSpec((1, _D, _H), lambda t, m: (m[0, t], 0, 0)),
            pl.BlockSpec((1, 1, _H), lambda t, m: (m[0, t], 0, 0)),
            pl.BlockSpec((1, _H, _O), lambda t, m: (m[0, t], 0, 0)),
            pl.BlockSpec((1, 1, _O), lambda t, m: (m[0, t], 0, 0)),
        ],
        out_specs=pl.BlockSpec((_T, _O), lambda t, m: (m[1, t], 0)),
    )
    return pl.pallas_call(
        _gmm_body,
        grid_spec=grid_spec,
        out_shape=jax.ShapeDtypeStruct((_PAD, _O), jnp.float32),
        interpret=interpret,
    )(meta, xs, w1, b1, w2, b2)


@functools.lru_cache(maxsize=None)
def _sc_kernels():
    mesh = plsc.VectorSubcoreMesh(core_axis_name="c", subcore_axis_name="s")

    @functools.partial(
        pl.kernel,
        out_type=jax.ShapeDtypeStruct((_PAD, _D // 2), jnp.int32),
        mesh=mesh,
        scratch_types=[
            pltpu.VMEM((_TOKW, _D // 2), jnp.int32),
            pltpu.VMEM((_K, _TOKW), jnp.int32),
            pltpu.SemaphoreType.DMA,
            pltpu.SemaphoreType.DMA,
        ],
    )
    def _sc_dispatch(xi_hbm, posw_hbm, out_hbm, xloc, idxs, s0, s1):
        wid = lax.axis_index("s") * _NC + lax.axis_index("c")
        pltpu.sync_copy(xi_hbm.at[pl.ds(wid * _TOKW, _TOKW)], xloc)
        pltpu.sync_copy(posw_hbm.at[wid], idxs)
        c0 = pltpu.async_copy(xloc, out_hbm.at[idxs.at[0]], s0)
        c1 = pltpu.async_copy(xloc, out_hbm.at[idxs.at[1]], s1)
        c0.wait()
        c1.wait()

    @functools.partial(
        pl.kernel,
        out_type=jax.ShapeDtypeStruct((_N, _O), jnp.float32),
        mesh=mesh,
        scratch_types=[
            pltpu.VMEM((_TOKW,), jnp.int32),
            pltpu.VMEM((_TOKW,), jnp.int32),
            pltpu.VMEM((_TOKW, 16), jnp.float32),
            pltpu.VMEM((_TOKW, 16), jnp.float32),
            pltpu.VMEM((_CCH, _O), jnp.float32),
            pltpu.VMEM((_CCH, _O), jnp.float32),
            pltpu.VMEM((_CCH, _O), jnp.float32),
            pltpu.VMEM((_CCH, _O), jnp.float32),
            pltpu.SemaphoreType.DMA,
            pltpu.SemaphoreType.DMA,
            pltpu.SemaphoreType.DMA,
            pltpu.SemaphoreType.DMA,
            pltpu.SemaphoreType.DMA,
            pltpu.SemaphoreType.DMA,
        ],
    )
    def _sc_combine(rows_hbm, pos0_hbm, pos1_hbm, gb0_hbm, gb1_hbm, y_hbm,
                    i0_v, i1_v, g0_v, g1_v, a0, a1, c0, c1,
                    ga0, ga1, gb0, gb1, wa, wb):
        wid = lax.axis_index("s") * _NC + lax.axis_index("c")
        base = wid * _TOKW
        pltpu.sync_copy(pos0_hbm.at[pl.ds(base, _TOKW)], i0_v)
        pltpu.sync_copy(pos1_hbm.at[pl.ds(base, _TOKW)], i1_v)
        pltpu.sync_copy(gb0_hbm.at[pl.ds(base, _TOKW)], g0_v)
        pltpu.sync_copy(gb1_hbm.at[pl.ds(base, _TOKW)], g1_v)
        slots = ((a0, a1, ga0, ga1, wa), (c0, c1, gb0, gb1, wb))
        nch = _TOKW // _CCH
        gathers = [None] * nch
        writes = [None] * nch

        def start(c):
            r0, r1, gs0, gs1, _ = slots[c % 2]
            sl = pl.ds(c * _CCH, _CCH)
            cp0 = pltpu.async_copy(rows_hbm.at[i0_v.at[sl]], r0, gs0)
            cp1 = pltpu.async_copy(rows_hbm.at[i1_v.at[sl]], r1, gs1)
            return (cp0, cp1)

        gathers[0] = start(0)
        for c in range(nch):
            r0, r1, _, _, ws = slots[c % 2]
            gathers[c][0].wait()
            gathers[c][1].wait()
            if c + 1 < nch:
                if c >= 1:
                    writes[c - 1].wait()
                gathers[c + 1] = start(c + 1)

            def row_body(r, _):
                tr = c * _CCH + r
                ga = g0_v[tr, :]
                gb = g1_v[tr, :]

                def vec_body(v, _):
                    sl = pl.ds(v * 16, 16)
                    r0[r, sl] = r0[r, sl] * ga + r1[r, sl] * gb
                    return 0

                return lax.fori_loop(0, _O // 16, vec_body, 0)

            lax.fori_loop(0, _CCH, row_body, 0)
            writes[c] = pltpu.async_copy(
                r0, y_hbm.at[pl.ds(base + c * _CCH, _CCH)], ws)
        if nch >= 2:
            writes[nch - 2].wait()
        writes[nch - 1].wait()

    return _sc_dispatch, _sc_combine


def _routing(i12, g12):
    e_flat = i12.reshape(-1)
    g_flat = g12.reshape(-1)
    oh = (e_flat[:, None] == jnp.arange(_E, dtype=jnp.int32)[None, :])
    oh = oh.astype(jnp.int32)
    csum = jnp.cumsum(oh, axis=0)
    counts = csum[-1]
    rank = jnp.sum(csum * oh, axis=1) - 1
    tiles = (counts + _T - 1) // _T
    zero1 = jnp.zeros((1,), jnp.int32)
    pad_start = jnp.concatenate([zero1, jnp.cumsum(tiles * _T)])
    dest = pad_start[e_flat] + rank
    pos = dest.reshape(_N, _K)
    nv = pad_start[_E] // _T
    tvec = jnp.arange(_NT, dtype=jnp.int32)
    eid = jnp.clip(jnp.searchsorted(pad_start[1:], tvec * _T, side="right"),
                   0, _E - 1).astype(jnp.int32)
    eid = jnp.where(tvec < nv, eid, eid[nv - 1])
    blk = jnp.where(tvec < nv, tvec, nv - 1)
    valid = (tvec < nv).astype(jnp.int32)
    meta = jnp.stack([eid, blk, valid])
    return pos, meta


def kernel(x, w_gate, W1, b1, W2, b2):
    wg_pad = jnp.zeros((_D, _EP), jnp.float32).at[:, :_E].set(w_gate)
    idx8, g8, loss8, xi32 = _gating_call(x, wg_pad)
    i12 = idx8[:, :_K]
    g12 = g8[:, :_K]
    pos, meta = _routing(i12, g12)
    sc_dispatch, sc_combine = _sc_kernels()
    posw = pos.reshape(_NW, _TOKW, _K).transpose(0, 2, 1)
    xs = sc_dispatch(xi32, posw)
    rows = _gmm_call(meta, xs, W1.astype(jnp.bfloat16), b1.reshape(_E, 1, _H),
                     W2.astype(jnp.bfloat16), b2.reshape(_E, 1, _O))
    gb0 = jnp.broadcast_to(g12[:, 0][:, None], (_N, 16))
    gb1 = jnp.broadcast_to(g12[:, 1][:, None], (_N, 16))
    y = sc_combine(rows, pos[:, 0], pos[:, 1], gb0, gb1)
    return y, loss8[0, 0]


# speedup vs baseline: 3.2728x; 1.1577x over previous
import functools

import jax
import jax.numpy as jnp
from jax import lax
from jax.experimental import pallas as pl
from jax.experimental.pallas import tpu as pltpu
from jax.experimental.pallas import tpu_sc as plsc

_N, _D, _H, _O, _E, _K = 2048, 1024, 1024, 1024, 8, 2
_S = _N * _K
_T = 256
_NT = (_S + _E * _T) // _T
_PAD = _NT * _T
_EP = 128

_NC, _NS = 2, 16
_NW = _NC * _NS
_DROWS = _PAD // _NW
_DCH = 48
_DSR = 16
_TOKW = _N // _NW
_CCH = 16


def _gating_body(x_ref, wg_ref, idx_ref, g_ref, loss_ref, xi_ref):
    x = x_ref[...]
    xb = x.astype(jnp.bfloat16)
    lo = lax.convert_element_type(
        lax.bitcast_convert_type(xb[:, : _D // 2], jnp.uint16), jnp.uint32)
    hi = lax.convert_element_type(
        lax.bitcast_convert_type(xb[:, _D // 2:], jnp.uint16), jnp.uint32)
    xi_ref[...] = lax.bitcast_convert_type((hi << 16) | lo, jnp.int32)
    wg = wg_ref[...]
    logits = jnp.dot(x, wg, preferred_element_type=jnp.float32)
    lane = lax.broadcasted_iota(jnp.int32, (_N, _EP), 1)
    neg = jnp.float32(-jnp.inf)
    logits = jnp.where(lane < _E, logits, neg)
    m1 = jnp.max(logits, axis=1, keepdims=True)
    i1 = jnp.min(jnp.where(logits == m1, lane, _EP), axis=1, keepdims=True)
    masked = jnp.where(lane == i1, neg, logits)
    m2 = jnp.max(masked, axis=1, keepdims=True)
    i2 = jnp.min(jnp.where(masked == m2, lane, _EP), axis=1, keepdims=True)
    t = jnp.exp(m2 - m1)
    g1 = 1.0 / (1.0 + t)
    g2 = t / (1.0 + t)
    gates = jnp.where(lane == i1, g1, 0.0) + jnp.where(lane == i2, g2, 0.0)
    gates = jnp.where(lane < _E, gates, 0.0)
    idx_ref[...] = jnp.where(lane == 0, i1, jnp.where(lane == 1, i2, 0))
    g_ref[...] = jnp.where(lane == 0, g1, jnp.where(lane == 1, g2, 0.0))
    emask = (lane[:1] < _E).astype(jnp.float32)
    imp = jnp.sum(gates, axis=0, keepdims=True)
    load = jnp.sum((gates > 0).astype(jnp.float32), axis=0, keepdims=True)

    def cv2(v):
        m = jnp.sum(v * emask) / _E
        var = jnp.sum((v - m) ** 2 * emask) / (_E - 1)
        return var / (m * m + 1e-10)

    loss = (cv2(imp) + cv2(load)) * 0.01
    loss_ref[...] = jnp.full((8, 128), loss, dtype=jnp.float32)


def _gating_call(x, wg_pad, interpret=False):
    return pl.pallas_call(
        _gating_body,
        out_shape=(
            jax.ShapeDtypeStruct((_N, _EP), jnp.int32),
            jax.ShapeDtypeStruct((_N, _EP), jnp.float32),
            jax.ShapeDtypeStruct((8, 128), jnp.float32),
            jax.ShapeDtypeStruct((_N, _D // 2), jnp.int32),
        ),
        interpret=interpret,
    )(x, wg_pad)


def _gmm_body(meta_ref, xs_ref, w1_ref, b1_ref, w2_ref, b2_ref, out_ref,
              w1b_ref, w2b_ref):
    t = pl.program_id(0)
    valid = meta_ref[2, t] == 1
    prev = jnp.maximum(t - 1, 0)
    changed = jnp.logical_or(t == 0, meta_ref[0, t] != meta_ref[0, prev])

    @pl.when(jnp.logical_and(valid, changed))
    def _():
        w1b_ref[...] = w1_ref[0].astype(jnp.bfloat16)
        w2b_ref[...] = w2_ref[0].astype(jnp.bfloat16)

    @pl.when(valid)
    def _():
        v = lax.bitcast_convert_type(xs_ref[...], jnp.uint32)
        lo = lax.bitcast_convert_type(
            lax.convert_element_type(v & 0xFFFF, jnp.uint16), jnp.bfloat16)
        hi = lax.bitcast_convert_type(
            lax.convert_element_type(v >> 16, jnp.uint16), jnp.bfloat16)
        xk = jnp.concatenate([lo, hi], axis=1)
        h = jnp.dot(xk, w1b_ref[...], preferred_element_type=jnp.float32)
        h = jnp.maximum(h + b1_ref[0], 0.0).astype(jnp.bfloat16)
        o = jnp.dot(h, w2b_ref[...], preferred_element_type=jnp.float32)
        out_ref[...] = o + b2_ref[0]


def _gmm_call(meta, xs, w1, b1, w2, b2, interpret=False):
    grid_spec = pltpu.PrefetchScalarGridSpec(
        num_scalar_prefetch=1,
        grid=(_NT,),
        in_specs=[
            pl.BlockSpec((_T, _D // 2), lambda t, m: (m[1, t], 0)),
            pl.BlockSpec((1, _D, _H), lambda t, m: (m[0, t], 0, 0)),
            pl.BlockSpec((1, 1, _H), lambda t, m: (m[0, t], 0, 0)),
            pl.BlockSpec((1, _H, _O), lambda t, m: (m[0, t], 0, 0)),
            pl.BlockSpec((1, 1, _O), lambda t, m: (m[0, t], 0, 0)),
        ],
        out_specs=pl.BlockSpec((_T, _O), lambda t, m: (m[1, t], 0)),
        scratch_shapes=[
            pltpu.VMEM((_D, _H), jnp.bfloat16),
            pltpu.VMEM((_H, _O), jnp.bfloat16),
        ],
    )
    return pl.pallas_call(
        _gmm_body,
        grid_spec=grid_spec,
        out_shape=jax.ShapeDtypeStruct((_PAD, _O), jnp.float32),
        interpret=interpret,
    )(meta, xs, w1, b1, w2, b2)


@functools.lru_cache(maxsize=None)
def _sc_kernels():
    mesh = plsc.VectorSubcoreMesh(core_axis_name="c", subcore_axis_name="s")

    @functools.partial(
        pl.kernel,
        out_type=jax.ShapeDtypeStruct((_PAD, _D // 2), jnp.int32),
        mesh=mesh,
        scratch_types=[
            pltpu.VMEM((_TOKW, _D // 2), jnp.int32),
            pltpu.VMEM((_K, _TOKW), jnp.int32),
            pltpu.SemaphoreType.DMA,
            pltpu.SemaphoreType.DMA,
        ],
    )
    def _sc_dispatch(xi_hbm, posw_hbm, out_hbm, xloc, idxs, s0, s1):
        wid = lax.axis_index("s") * _NC + lax.axis_index("c")
        pltpu.sync_copy(xi_hbm.at[pl.ds(wid * _TOKW, _TOKW)], xloc)
        pltpu.sync_copy(posw_hbm.at[wid], idxs)
        c0 = pltpu.async_copy(xloc, out_hbm.at[idxs.at[0]], s0)
        c1 = pltpu.async_copy(xloc, out_hbm.at[idxs.at[1]], s1)
        c0.wait()
        c1.wait()

    @functools.partial(
        pl.kernel,
        out_type=jax.ShapeDtypeStruct((_N, _O), jnp.float32),
        mesh=mesh,
        scratch_types=[
            pltpu.VMEM((_TOKW,), jnp.int32),
            pltpu.VMEM((_TOKW,), jnp.int32),
            pltpu.VMEM((_TOKW, 16), jnp.float32),
            pltpu.VMEM((_TOKW, 16), jnp.float32),
            pltpu.VMEM((_CCH, _O), jnp.float32),
            pltpu.VMEM((_CCH, _O), jnp.float32),
            pltpu.VMEM((_CCH, _O), jnp.float32),
            pltpu.VMEM((_CCH, _O), jnp.float32),
            pltpu.SemaphoreType.DMA,
            pltpu.SemaphoreType.DMA,
            pltpu.SemaphoreType.DMA,
            pltpu.SemaphoreType.DMA,
            pltpu.SemaphoreType.DMA,
            pltpu.SemaphoreType.DMA,
        ],
    )
    def _sc_combine(rows_hbm, pos0_hbm, pos1_hbm, gb0_hbm, gb1_hbm, y_hbm,
                    i0_v, i1_v, g0_v, g1_v, a0, a1, c0, c1,
                    ga0, ga1, gb0, gb1, wa, wb):
        wid = lax.axis_index("s") * _NC + lax.axis_index("c")
        base = wid * _TOKW
        pltpu.sync_copy(pos0_hbm.at[pl.ds(base, _TOKW)], i0_v)
        pltpu.sync_copy(pos1_hbm.at[pl.ds(base, _TOKW)], i1_v)
        pltpu.sync_copy(gb0_hbm.at[pl.ds(base, _TOKW)], g0_v)
        pltpu.sync_copy(gb1_hbm.at[pl.ds(base, _TOKW)], g1_v)
        slots = ((a0, a1, ga0, ga1, wa), (c0, c1, gb0, gb1, wb))
        nch = _TOKW // _CCH
        gathers = [None] * nch
        writes = [None] * nch

        def start(c):
            r0, r1, gs0, gs1, _ = slots[c % 2]
            sl = pl.ds(c * _CCH, _CCH)
            cp0 = pltpu.async_copy(rows_hbm.at[i0_v.at[sl]], r0, gs0)
            cp1 = pltpu.async_copy(rows_hbm.at[i1_v.at[sl]], r1, gs1)
            return (cp0, cp1)

        gathers[0] = start(0)
        for c in range(nch):
            r0, r1, _, _, ws = slots[c % 2]
            gathers[c][0].wait()
            gathers[c][1].wait()
            if c + 1 < nch:
                if c >= 1:
                    writes[c - 1].wait()
                gathers[c + 1] = start(c + 1)

            def row_body(r, _):
                tr = c * _CCH + r
                ga = g0_v[tr, :]
                gb = g1_v[tr, :]

                def vec_body(v, _):
                    sl = pl.ds(v * 16, 16)
                    r0[r, sl] = r0[r, sl] * ga + r1[r, sl] * gb
                    return 0

                return lax.fori_loop(0, _O // 16, vec_body, 0)

            lax.fori_loop(0, _CCH, row_body, 0)
            writes[c] = pltpu.async_copy(
                r0, y_hbm.at[pl.ds(base + c * _CCH, _CCH)], ws)
        if nch >= 2:
            writes[nch - 2].wait()
        writes[nch - 1].wait()

    return _sc_dispatch, _sc_combine


def _routing(i12, g12):
    e_flat = i12.reshape(-1)
    g_flat = g12.reshape(-1)
    oh = (e_flat[:, None] == jnp.arange(_E, dtype=jnp.int32)[None, :])
    oh = oh.astype(jnp.int32)
    csum = jnp.cumsum(oh, axis=0)
    counts = csum[-1]
    rank = jnp.sum(csum * oh, axis=1) - 1
    tiles = (counts + _T - 1) // _T
    zero1 = jnp.zeros((1,), jnp.int32)
    pad_start = jnp.concatenate([zero1, jnp.cumsum(tiles * _T)])
    dest = pad_start[e_flat] + rank
    pos = dest.reshape(_N, _K)
    nv = pad_start[_E] // _T
    tvec = jnp.arange(_NT, dtype=jnp.int32)
    eid = jnp.clip(jnp.searchsorted(pad_start[1:], tvec * _T, side="right"),
                   0, _E - 1).astype(jnp.int32)
    eid = jnp.where(tvec < nv, eid, eid[nv - 1])
    blk = jnp.where(tvec < nv, tvec, nv - 1)
    valid = (tvec < nv).astype(jnp.int32)
    meta = jnp.stack([eid, blk, valid])
    return pos, meta


def kernel(x, w_gate, W1, b1, W2, b2):
    wg_pad = jnp.zeros((_D, _EP), jnp.float32).at[:, :_E].set(w_gate)
    idx8, g8, loss8, xi32 = _gating_call(x, wg_pad)
    i12 = idx8[:, :_K]
    g12 = g8[:, :_K]
    pos, meta = _routing(i12, g12)
    sc_dispatch, sc_combine = _sc_kernels()
    posw = pos.reshape(_NW, _TOKW, _K).transpose(0, 2, 1)
    xs = sc_dispatch(xi32, posw)
    rows = _gmm_call(meta, xs, W1, b1.reshape(_E, 1, _H),
                     W2, b2.reshape(_E, 1, _O))
    gb0 = jnp.broadcast_to(g12[:, 0][:, None], (_N, 16))
    gb1 = jnp.broadcast_to(g12[:, 1][:, None], (_N, 16))
    y = sc_combine(rows, pos[:, 0], pos[:, 1], gb0, gb1)
    return y, loss8[0, 0]


# speedup vs baseline: 3.4676x; 1.0595x over previous
import functools

import jax
import jax.numpy as jnp
from jax import lax
from jax.experimental import pallas as pl
from jax.experimental.pallas import tpu as pltpu
from jax.experimental.pallas import tpu_sc as plsc

_N, _D, _H, _O, _E, _K = 2048, 1024, 1024, 1024, 8, 2
_S = _N * _K
_T = 512
_NT = -(-(_S + _E * (_T - 1)) // _T)
_PAD = _NT * _T
_EP = 128

_NC, _NS = 2, 16
_NW = _NC * _NS
_DROWS = _PAD // _NW
_DCH = 48
_DSR = 16
_TOKW = _N // _NW
_CCH = 16


def _gating_body(x_ref, wg_ref, idx_ref, g_ref, loss_ref, xi_ref):
    x = x_ref[...]
    xb = x.astype(jnp.bfloat16)
    lo = lax.convert_element_type(
        lax.bitcast_convert_type(xb[:, : _D // 2], jnp.uint16), jnp.uint32)
    hi = lax.convert_element_type(
        lax.bitcast_convert_type(xb[:, _D // 2:], jnp.uint16), jnp.uint32)
    xi_ref[...] = lax.bitcast_convert_type((hi << 16) | lo, jnp.int32)
    wg = wg_ref[...]
    logits = jnp.dot(x, wg, preferred_element_type=jnp.float32)
    lane = lax.broadcasted_iota(jnp.int32, (_N, _EP), 1)
    neg = jnp.float32(-jnp.inf)
    logits = jnp.where(lane < _E, logits, neg)
    m1 = jnp.max(logits, axis=1, keepdims=True)
    i1 = jnp.min(jnp.where(logits == m1, lane, _EP), axis=1, keepdims=True)
    masked = jnp.where(lane == i1, neg, logits)
    m2 = jnp.max(masked, axis=1, keepdims=True)
    i2 = jnp.min(jnp.where(masked == m2, lane, _EP), axis=1, keepdims=True)
    t = jnp.exp(m2 - m1)
    g1 = 1.0 / (1.0 + t)
    g2 = t / (1.0 + t)
    gates = jnp.where(lane == i1, g1, 0.0) + jnp.where(lane == i2, g2, 0.0)
    gates = jnp.where(lane < _E, gates, 0.0)
    idx_ref[...] = jnp.where(lane == 0, i1, jnp.where(lane == 1, i2, 0))
    g_ref[...] = jnp.where(lane == 0, g1, jnp.where(lane == 1, g2, 0.0))
    emask = (lane[:1] < _E).astype(jnp.float32)
    imp = jnp.sum(gates, axis=0, keepdims=True)
    load = jnp.sum((gates > 0).astype(jnp.float32), axis=0, keepdims=True)

    def cv2(v):
        m = jnp.sum(v * emask) / _E
        var = jnp.sum((v - m) ** 2 * emask) / (_E - 1)
        return var / (m * m + 1e-10)

    loss = (cv2(imp) + cv2(load)) * 0.01
    loss_ref[...] = jnp.full((8, 128), loss, dtype=jnp.float32)


def _gating_call(x, wg_pad, interpret=False):
    return pl.pallas_call(
        _gating_body,
        out_shape=(
            jax.ShapeDtypeStruct((_N, _EP), jnp.int32),
            jax.ShapeDtypeStruct((_N, _EP), jnp.float32),
            jax.ShapeDtypeStruct((8, 128), jnp.float32),
            jax.ShapeDtypeStruct((_N, _D // 2), jnp.int32),
        ),
        interpret=interpret,
    )(x, wg_pad)


def _gmm_body(meta_ref, xs_ref, w1_ref, b1_ref, w2_ref, b2_ref, out_ref,
              w1b_ref, w2b_ref):
    t = pl.program_id(0)
    valid = meta_ref[2, t] == 1
    prev = jnp.maximum(t - 1, 0)
    changed = jnp.logical_or(t == 0, meta_ref[0, t] != meta_ref[0, prev])

    @pl.when(jnp.logical_and(valid, changed))
    def _():
        w1b_ref[...] = w1_ref[0].astype(jnp.bfloat16)
        w2b_ref[...] = w2_ref[0].astype(jnp.bfloat16)

    @pl.when(valid)
    def _():
        v = lax.bitcast_convert_type(xs_ref[...], jnp.uint32)
        lo = lax.bitcast_convert_type(
            lax.convert_element_type(v & 0xFFFF, jnp.uint16), jnp.bfloat16)
        hi = lax.bitcast_convert_type(
            lax.convert_element_type(v >> 16, jnp.uint16), jnp.bfloat16)
        xk = jnp.concatenate([lo, hi], axis=1)
        h = jnp.dot(xk, w1b_ref[...], preferred_element_type=jnp.float32)
        h = jnp.maximum(h + b1_ref[0], 0.0).astype(jnp.bfloat16)
        o = jnp.dot(h, w2b_ref[...], preferred_element_type=jnp.float32)
        out_ref[...] = o + b2_ref[0]


def _gmm_call(meta, xs, w1, b1, w2, b2, interpret=False):
    grid_spec = pltpu.PrefetchScalarGridSpec(
        num_scalar_prefetch=1,
        grid=(_NT,),
        in_specs=[
            pl.BlockSpec((_T, _D // 2), lambda t, m: (m[1, t], 0)),
            pl.BlockSpec((1, _D, _H), lambda t, m: (m[0, t], 0, 0)),
            pl.BlockSpec((1, 1, _H), lambda t, m: (m[0, t], 0, 0)),
            pl.BlockSpec((1, _H, _O), lambda t, m: (m[0, t], 0, 0)),
            pl.BlockSpec((1, 1, _O), lambda t, m: (m[0, t], 0, 0)),
        ],
        out_specs=pl.BlockSpec((_T, _O), lambda t, m: (m[1, t], 0)),
        scratch_shapes=[
            pltpu.VMEM((_D, _H), jnp.bfloat16),
            pltpu.VMEM((_H, _O), jnp.bfloat16),
        ],
    )
    return pl.pallas_call(
        _gmm_body,
        grid_spec=grid_spec,
        out_shape=jax.ShapeDtypeStruct((_PAD, _O), jnp.float32),
        interpret=interpret,
    )(meta, xs, w1, b1, w2, b2)


@functools.lru_cache(maxsize=None)
def _sc_kernels():
    mesh = plsc.VectorSubcoreMesh(core_axis_name="c", subcore_axis_name="s")

    @functools.partial(
        pl.kernel,
        out_type=jax.ShapeDtypeStruct((_PAD, _D // 2), jnp.int32),
        mesh=mesh,
        scratch_types=[
            pltpu.VMEM((_TOKW, _D // 2), jnp.int32),
            pltpu.VMEM((_K, _TOKW), jnp.int32),
            pltpu.SemaphoreType.DMA,
            pltpu.SemaphoreType.DMA,
        ],
    )
    def _sc_dispatch(xi_hbm, posw_hbm, out_hbm, xloc, idxs, s0, s1):
        wid = lax.axis_index("s") * _NC + lax.axis_index("c")
        pltpu.sync_copy(xi_hbm.at[pl.ds(wid * _TOKW, _TOKW)], xloc)
        pltpu.sync_copy(posw_hbm.at[wid], idxs)
        c0 = pltpu.async_copy(xloc, out_hbm.at[idxs.at[0]], s0)
        c1 = pltpu.async_copy(xloc, out_hbm.at[idxs.at[1]], s1)
        c0.wait()
        c1.wait()

    @functools.partial(
        pl.kernel,
        out_type=jax.ShapeDtypeStruct((_N, _O), jnp.float32),
        mesh=mesh,
        scratch_types=[
            pltpu.VMEM((_TOKW,), jnp.int32),
            pltpu.VMEM((_TOKW,), jnp.int32),
            pltpu.VMEM((_TOKW, 16), jnp.float32),
            pltpu.VMEM((_TOKW, 16), jnp.float32),
            pltpu.VMEM((_CCH, _O), jnp.float32),
            pltpu.VMEM((_CCH, _O), jnp.float32),
            pltpu.VMEM((_CCH, _O), jnp.float32),
            pltpu.VMEM((_CCH, _O), jnp.float32),
            pltpu.VMEM((_CCH, _O), jnp.float32),
            pltpu.VMEM((_CCH, _O), jnp.float32),
            pltpu.SemaphoreType.DMA,
            pltpu.SemaphoreType.DMA,
            pltpu.SemaphoreType.DMA,
            pltpu.SemaphoreType.DMA,
            pltpu.SemaphoreType.DMA,
            pltpu.SemaphoreType.DMA,
            pltpu.SemaphoreType.DMA,
            pltpu.SemaphoreType.DMA,
            pltpu.SemaphoreType.DMA,
        ],
    )
    def _sc_combine(rows_hbm, pos0_hbm, pos1_hbm, gb0_hbm, gb1_hbm, y_hbm,
                    i0_v, i1_v, g0_v, g1_v, a0, a1, c0, c1, e0, e1,
                    ga0, ga1, gb0, gb1, gc0, gc1, wa, wb, wc):
        wid = lax.axis_index("s") * _NC + lax.axis_index("c")
        base = wid * _TOKW
        pltpu.sync_copy(pos0_hbm.at[pl.ds(base, _TOKW)], i0_v)
        pltpu.sync_copy(pos1_hbm.at[pl.ds(base, _TOKW)], i1_v)
        pltpu.sync_copy(gb0_hbm.at[pl.ds(base, _TOKW)], g0_v)
        pltpu.sync_copy(gb1_hbm.at[pl.ds(base, _TOKW)], g1_v)
        slots = ((a0, a1, ga0, ga1, wa), (c0, c1, gb0, gb1, wb),
                 (e0, e1, gc0, gc1, wc))
        nch = _TOKW // _CCH
        gathers = [None] * nch
        writes = [None] * nch

        def start(c):
            r0, r1, gs0, gs1, _ = slots[c % 3]
            sl = pl.ds(c * _CCH, _CCH)
            cp0 = pltpu.async_copy(rows_hbm.at[i0_v.at[sl]], r0, gs0)
            cp1 = pltpu.async_copy(rows_hbm.at[i1_v.at[sl]], r1, gs1)
            return (cp0, cp1)

        gathers[0] = start(0)
        gathers[1] = start(1)
        for c in range(nch):
            r0, r1, _, _, ws = slots[c % 3]
            gathers[c][0].wait()
            gathers[c][1].wait()
            if c + 2 < nch:
                if c + 2 >= 3:
                    writes[c - 1].wait()
                gathers[c + 2] = start(c + 2)

            def row_body(r, _):
                tr = c * _CCH + r
                ga = g0_v[tr, :]
                gb = g1_v[tr, :]

                def vec_body(v, _):
                    sl = pl.ds(v * 16, 16)
                    r0[r, sl] = r0[r, sl] * ga + r1[r, sl] * gb
                    return 0

                return lax.fori_loop(0, _O // 16, vec_body, 0)

            lax.fori_loop(0, _CCH, row_body, 0)
            writes[c] = pltpu.async_copy(
                r0, y_hbm.at[pl.ds(base + c * _CCH, _CCH)], ws)
        for c in range(max(0, nch - 3), nch):
            writes[c].wait()

    return _sc_dispatch, _sc_combine


def _routing(i12, g12):
    e_flat = i12.reshape(-1)
    g_flat = g12.reshape(-1)
    oh = (e_flat[:, None] == jnp.arange(_E, dtype=jnp.int32)[None, :])
    oh = oh.astype(jnp.int32)
    csum = jnp.cumsum(oh, axis=0)
    counts = csum[-1]
    rank = jnp.sum(csum * oh, axis=1) - 1
    tiles = (counts + _T - 1) // _T
    zero1 = jnp.zeros((1,), jnp.int32)
    pad_start = jnp.concatenate([zero1, jnp.cumsum(tiles * _T)])
    dest = pad_start[e_flat] + rank
    pos = dest.reshape(_N, _K)
    nv = pad_start[_E] // _T
    tvec = jnp.arange(_NT, dtype=jnp.int32)
    eid = jnp.clip(jnp.searchsorted(pad_start[1:], tvec * _T, side="right"),
                   0, _E - 1).astype(jnp.int32)
    eid = jnp.where(tvec < nv, eid, eid[nv - 1])
    blk = jnp.where(tvec < nv, tvec, nv - 1)
    valid = (tvec < nv).astype(jnp.int32)
    meta = jnp.stack([eid, blk, valid])
    return pos, meta


def kernel(x, w_gate, W1, b1, W2, b2):
    wg_pad = jnp.zeros((_D, _EP), jnp.float32).at[:, :_E].set(w_gate)
    idx8, g8, loss8, xi32 = _gating_call(x, wg_pad)
    i12 = idx8[:, :_K]
    g12 = g8[:, :_K]
    pos, meta = _routing(i12, g12)
    sc_dispatch, sc_combine = _sc_kernels()
    posw = pos.reshape(_NW, _TOKW, _K).transpose(0, 2, 1)
    xs = sc_dispatch(xi32, posw)
    rows = _gmm_call(meta, xs, W1, b1.reshape(_E, 1, _H),
                     W2, b2.reshape(_E, 1, _O))
    gb0 = jnp.broadcast_to(g12[:, 0][:, None], (_N, 16))
    gb1 = jnp.broadcast_to(g12[:, 1][:, None], (_N, 16))
    y = sc_combine(rows, pos[:, 0], pos[:, 1], gb0, gb1)
    return y, loss8[0, 0]
